# baseline (speedup 1.0000x reference)
import functools
import math

import jax
import jax.numpy as jnp
import numpy as np
from jax import lax
from jax.experimental import pallas as pl
from jax.experimental.pallas import tpu as pltpu
from jax.experimental.pallas import tpu_sc as plsc

SEQ_LEN = 2048
DIM = 128
BATCH = 16
SCALE = math.sqrt(float(DIM))

NUM_CORES = 2
NUM_SUBCORES = 16
NW = NUM_CORES * NUM_SUBCORES
B_TOTAL = BATCH * SEQ_LEN
B_PER_W = B_TOTAL // NW
CHUNK = 128
N_CHUNKS = B_PER_W // CHUNK
LANES = 16
VECS_PER_ROW = DIM // LANES


def _positional_encoding_np():
    n = 10000.0
    pos = np.arange(SEQ_LEN, dtype=np.float64)[:, None]
    i = np.arange(DIM // 2, dtype=np.float64)[None, :]
    denom = n ** (2.0 * i / DIM)
    enc = np.zeros((SEQ_LEN, DIM), dtype=np.float32)
    enc[:, 0::2] = np.sin(pos / denom).astype(np.float32)
    enc[:, 1::2] = np.cos(pos / denom).astype(np.float32)
    return enc


_POS_ENC = _positional_encoding_np()


def _embed_body(table_hbm, idx_hbm, pos_hbm, out_hbm, idx_v, rows_v, pos_v, sem):
    wid = lax.axis_index("s") * NUM_CORES + lax.axis_index("c")
    base = wid * B_PER_W
    pos_base = (wid % 2) * B_PER_W

    def step(c, _):
        off = c * CHUNK
        pltpu.sync_copy(idx_hbm.at[pl.ds(base + off, CHUNK)], idx_v)
        pltpu.sync_copy(pos_hbm.at[pl.ds(pos_base + off, CHUNK)], pos_v)
        pltpu.async_copy(table_hbm.at[idx_v], rows_v, sem).wait()

        def fma_row(r, _):
            for j in range(VECS_PER_ROW):
                sl = pl.ds(j * LANES, LANES)
                rows_v[r, sl] = rows_v[r, sl] * SCALE + pos_v[r, sl]
            return 0

        lax.fori_loop(0, CHUNK, fma_row, 0, unroll=2)
        pltpu.sync_copy(rows_v, out_hbm.at[pl.ds(base + off, CHUNK)])
        return 0

    lax.fori_loop(0, N_CHUNKS, step, 0)


@jax.jit
def _embed(idx_flat, table):
    pos_enc = jnp.asarray(_POS_ENC)
    mesh = plsc.VectorSubcoreMesh(
        core_axis_name="c", subcore_axis_name="s",
        num_cores=NUM_CORES, num_subcores=NUM_SUBCORES)
    fn = pl.kernel(
        _embed_body,
        out_type=jax.ShapeDtypeStruct((B_TOTAL, DIM), jnp.float32),
        mesh=mesh,
        scratch_types=[
            pltpu.VMEM((CHUNK,), jnp.int32),
            pltpu.VMEM((CHUNK, DIM), jnp.float32),
            pltpu.VMEM((CHUNK, DIM), jnp.float32),
            pltpu.SemaphoreType.DMA,
        ],
    )
    return fn(table, idx_flat, pos_enc)


def kernel(inputs, table):
    idx_flat = inputs.reshape(-1).astype(jnp.int32)
    out = _embed(idx_flat, table)
    return out.reshape(BATCH, SEQ_LEN, DIM)


# speedup vs baseline: 2.5311x; 2.5311x over previous
import math

import jax
import jax.numpy as jnp
import numpy as np
from jax import lax
from jax.experimental import pallas as pl
from jax.experimental.pallas import tpu as pltpu
from jax.experimental.pallas import tpu_sc as plsc

SEQ_LEN = 2048
DIM = 128
BATCH = 16
SCALE = math.sqrt(float(DIM))

NUM_CORES = 2
NUM_SUBCORES = 16
NW = NUM_CORES * NUM_SUBCORES
P_PER_W = SEQ_LEN // NW
B_PER_STEP = 4
N_STEPS = BATCH // B_PER_STEP
ROWS_PER_STEP = B_PER_STEP * P_PER_W
NBUF = 3
LANES = 16
VECS_PER_ROW = DIM // LANES


def _positional_encoding_np():
    n = 10000.0
    pos = np.arange(SEQ_LEN, dtype=np.float64)[:, None]
    i = np.arange(DIM // 2, dtype=np.float64)[None, :]
    denom = n ** (2.0 * i / DIM)
    enc = np.zeros((SEQ_LEN, DIM), dtype=np.float32)
    enc[:, 0::2] = np.sin(pos / denom).astype(np.float32)
    enc[:, 1::2] = np.cos(pos / denom).astype(np.float32)
    return enc


_POS_ENC = _positional_encoding_np()


def _embed_body(table_hbm, idx_hbm, pos_hbm, out_hbm,
                idx_v, pos_v, b0, b1, b2, gs0, gs1, gs2, ws0, ws1, ws2):
    bufs = [b0, b1, b2]
    gsems = [gs0, gs1, gs2]
    wsems = [ws0, ws1, ws2]
    wid = lax.axis_index("s") * NUM_CORES + lax.axis_index("c")
    base = wid * (BATCH * P_PER_W)
    pbase = wid * P_PER_W

    pltpu.sync_copy(idx_hbm.at[pl.ds(base, BATCH * P_PER_W)], idx_v)
    pltpu.sync_copy(pos_hbm.at[pl.ds(pbase, P_PER_W)], pos_v)

    def start_gather(s):
        idx_slice = idx_v.at[pl.ds(s * ROWS_PER_STEP, ROWS_PER_STEP)]
        return pltpu.async_copy(table_hbm.at[idx_slice], bufs[s % NBUF],
                                gsems[s % NBUF])

    gather_h = {0: start_gather(0)}
    write_h = {}

    for s in range(N_STEPS):
        buf = bufs[s % NBUF]
        gather_h.pop(s).wait()
        if s + 1 < N_STEPS:
            for h in write_h.pop(s + 1 - NBUF, ()):
                h.wait()
            gather_h[s + 1] = start_gather(s + 1)

        def fma_pos(p, _, buf=buf):
            for j in range(VECS_PER_ROW):
                sl = pl.ds(j * LANES, LANES)
                pv = pos_v[p, sl]
                for bb in range(B_PER_STEP):
                    r = bb * P_PER_W + p
                    buf[r, sl] = buf[r, sl] * SCALE + pv
            return 0

        for h in write_h.pop(s - NBUF, ()):
            h.wait()
        lax.fori_loop(0, P_PER_W, fma_pos, 0)

        hs = []
        for bb in range(B_PER_STEP):
            b = s * B_PER_STEP + bb
            hs.append(pltpu.async_copy(
                buf.at[pl.ds(bb * P_PER_W, P_PER_W)],
                out_hbm.at[b, pl.ds(pbase, P_PER_W)],
                wsems[s % NBUF]))
        write_h[s] = hs

    for hs in write_h.values():
        for h in hs:
            h.wait()


@jax.jit
def _embed(idx_bm, table):
    pos_enc = jnp.asarray(_POS_ENC)
    mesh = plsc.VectorSubcoreMesh(
        core_axis_name="c", subcore_axis_name="s",
        num_cores=NUM_CORES, num_subcores=NUM_SUBCORES)
    fn = pl.kernel(
        _embed_body,
        out_type=jax.ShapeDtypeStruct((BATCH, SEQ_LEN, DIM), jnp.float32),
        mesh=mesh,
        scratch_types=[
            pltpu.VMEM((BATCH * P_PER_W,), jnp.int32),
            pltpu.VMEM((P_PER_W, DIM), jnp.float32),
            pltpu.VMEM((ROWS_PER_STEP, DIM), jnp.float32),
            pltpu.VMEM((ROWS_PER_STEP, DIM), jnp.float32),
            pltpu.VMEM((ROWS_PER_STEP, DIM), jnp.float32),
            pltpu.SemaphoreType.DMA,
            pltpu.SemaphoreType.DMA,
            pltpu.SemaphoreType.DMA,
            pltpu.SemaphoreType.DMA,
            pltpu.SemaphoreType.DMA,
            pltpu.SemaphoreType.DMA,
        ],
    )
    return fn(table, idx_bm, pos_enc)


def kernel(inputs, table):
    idx_bm = (inputs.astype(jnp.int32)
              .reshape(BATCH, NW, P_PER_W)
              .transpose(1, 0, 2)
              .reshape(-1))
    return _embed(idx_bm, table)
